# baseline (speedup 1.0000x reference)
import functools

import jax
import jax.numpy as jnp
from jax.experimental import pallas as pl


def _fused_kernel(x_ref, mask_ref, starts_ref, ends_ref, w1h_ref, w1l_ref,
                  b1_ref, w2_ref, b2_ref, out_ref):
    x = x_ref[0] * mask_ref[0]
    xh = x.astype(jnp.bfloat16)
    xl = (x - xh.astype(jnp.float32)).astype(jnp.bfloat16)
    wh = w1h_ref[...]
    h = (jnp.dot(wh, xh, preferred_element_type=jnp.float32)
         + jnp.dot(wh, xl, preferred_element_type=jnp.float32)
         + jnp.dot(w1l_ref[...], xh, preferred_element_type=jnp.float32))
    h = jnp.maximum(h + b1_ref[0][:, None], 0.0)
    s = jnp.dot(w2_ref[...], h, preferred_element_type=jnp.float32)
    s = s + b2_ref[0, 0]

    t = jax.lax.broadcasted_iota(jnp.int32, (starts_ref.shape[-1], s.shape[-1]), 1)
    starts = starts_ref[0, 0, :][:, None]
    ends = ends_ref[0, 0, :][:, None]
    in_word = (t >= starts) & (t < ends)
    masked = jnp.where(in_word, s, -jnp.inf)
    out_ref[0, 0, :] = jnp.max(masked, axis=-1)


def kernel(features, word_bounds, word_lengths, mask, W1, b1, W2, b2):
    B, D, T = features.shape
    H = W1.shape[0]
    W = word_bounds.shape[-1]

    starts = word_bounds[:, 0, :].astype(jnp.int32).reshape(B, 1, W)
    ends = word_bounds[:, 1, :].astype(jnp.int32).reshape(B, 1, W)
    b1r = b1.reshape(1, H).astype(jnp.float32)
    b2r = b2.reshape(1, 1).astype(jnp.float32)
    W1h = W1.astype(jnp.bfloat16)
    W1l = (W1 - W1h.astype(jnp.float32)).astype(jnp.bfloat16)

    out = pl.pallas_call(
        _fused_kernel,
        grid=(B,),
        in_specs=[
            pl.BlockSpec((1, D, T), lambda b: (b, 0, 0)),
            pl.BlockSpec((1, 1, T), lambda b: (b, 0, 0)),
            pl.BlockSpec((1, 1, W), lambda b: (b, 0, 0)),
            pl.BlockSpec((1, 1, W), lambda b: (b, 0, 0)),
            pl.BlockSpec((H, D), lambda b: (0, 0)),
            pl.BlockSpec((H, D), lambda b: (0, 0)),
            pl.BlockSpec((1, H), lambda b: (0, 0)),
            pl.BlockSpec((1, H), lambda b: (0, 0)),
            pl.BlockSpec((1, 1), lambda b: (0, 0)),
        ],
        out_specs=pl.BlockSpec((1, 1, W), lambda b: (b, 0, 0)),
        out_shape=jax.ShapeDtypeStruct((B, 1, W), jnp.float32),
    )(features, mask, starts, ends, W1h, W1l, b1r, W2, b2r)
    return out


# speedup vs baseline: 2.0203x; 2.0203x over previous
import functools

import jax
import jax.numpy as jnp
from jax.experimental import pallas as pl


def _fused_kernel(x_ref, mask_ref, starts_ref, ends_ref, w1_ref, b1_ref,
                  w2_ref, b2_ref, out_ref):
    x = x_ref[0] * mask_ref[0]
    h = jnp.dot(w1_ref[...], x, preferred_element_type=jnp.float32)
    h = jnp.maximum(h + b1_ref[0][:, None], 0.0)
    s = jnp.dot(w2_ref[...], h, preferred_element_type=jnp.float32)
    s = s + b2_ref[0, 0]

    t = jax.lax.broadcasted_iota(jnp.int32, (starts_ref.shape[-1], s.shape[-1]), 1)
    starts = starts_ref[0, 0, :][:, None]
    ends = ends_ref[0, 0, :][:, None]
    in_word = (t >= starts) & (t < ends)
    masked = jnp.where(in_word, s, -jnp.inf)
    out_ref[0, 0, :] = jnp.max(masked, axis=-1)


def kernel(features, word_bounds, word_lengths, mask, W1, b1, W2, b2):
    B, D, T = features.shape
    H = W1.shape[0]
    W = word_bounds.shape[-1]

    starts = word_bounds[:, 0, :].astype(jnp.int32).reshape(B, 1, W)
    ends = word_bounds[:, 1, :].astype(jnp.int32).reshape(B, 1, W)
    b1r = b1.reshape(1, H).astype(jnp.float32)
    b2r = b2.reshape(1, 1).astype(jnp.float32)

    out = pl.pallas_call(
        _fused_kernel,
        grid=(B,),
        in_specs=[
            pl.BlockSpec((1, D, T), lambda b: (b, 0, 0)),
            pl.BlockSpec((1, 1, T), lambda b: (b, 0, 0)),
            pl.BlockSpec((1, 1, W), lambda b: (b, 0, 0)),
            pl.BlockSpec((1, 1, W), lambda b: (b, 0, 0)),
            pl.BlockSpec((H, D), lambda b: (0, 0)),
            pl.BlockSpec((1, H), lambda b: (0, 0)),
            pl.BlockSpec((1, H), lambda b: (0, 0)),
            pl.BlockSpec((1, 1), lambda b: (0, 0)),
        ],
        out_specs=pl.BlockSpec((1, 1, W), lambda b: (b, 0, 0)),
        out_shape=jax.ShapeDtypeStruct((B, 1, W), jnp.float32),
    )(features, mask, starts, ends, W1, b1r, W2, b2r)
    return out


# speedup vs baseline: 2.0330x; 1.0062x over previous
import functools

import jax
import jax.numpy as jnp
from jax.experimental import pallas as pl
from jax.experimental.pallas import tpu as pltpu


def _fused_kernel(x_ref, mask_ref, starts_ref, ends_ref, w1_ref, b1_ref,
                  w2_ref, b2_ref, out_ref):
    x = x_ref[0] * mask_ref[0]
    h = jnp.dot(w1_ref[...], x, preferred_element_type=jnp.float32)
    h = jnp.maximum(h + b1_ref[0][:, None], 0.0)
    s = jnp.dot(w2_ref[...], h, preferred_element_type=jnp.float32)
    s = s + b2_ref[0, 0]

    t = jax.lax.broadcasted_iota(jnp.int32, (starts_ref.shape[-1], s.shape[-1]), 1)
    starts = starts_ref[0, 0, :][:, None]
    ends = ends_ref[0, 0, :][:, None]
    in_word = (t >= starts) & (t < ends)
    masked = jnp.where(in_word, s, -jnp.inf)
    out_ref[0, 0, :] = jnp.max(masked, axis=-1)


def kernel(features, word_bounds, word_lengths, mask, W1, b1, W2, b2):
    B, D, T = features.shape
    H = W1.shape[0]
    W = word_bounds.shape[-1]

    starts = word_bounds[:, 0, :].astype(jnp.int32).reshape(B, 1, W)
    ends = word_bounds[:, 1, :].astype(jnp.int32).reshape(B, 1, W)
    b1r = b1.reshape(1, H).astype(jnp.float32)
    b2r = b2.reshape(1, 1).astype(jnp.float32)

    out = pl.pallas_call(
        _fused_kernel,
        grid=(B,),
        in_specs=[
            pl.BlockSpec((1, D, T), lambda b: (b, 0, 0)),
            pl.BlockSpec((1, 1, T), lambda b: (b, 0, 0)),
            pl.BlockSpec((1, 1, W), lambda b: (b, 0, 0)),
            pl.BlockSpec((1, 1, W), lambda b: (b, 0, 0)),
            pl.BlockSpec((H, D), lambda b: (0, 0)),
            pl.BlockSpec((1, H), lambda b: (0, 0)),
            pl.BlockSpec((1, H), lambda b: (0, 0)),
            pl.BlockSpec((1, 1), lambda b: (0, 0)),
        ],
        out_specs=pl.BlockSpec((1, 1, W), lambda b: (b, 0, 0)),
        out_shape=jax.ShapeDtypeStruct((B, 1, W), jnp.float32),
        compiler_params=pltpu.CompilerParams(
            dimension_semantics=("parallel",)),
    )(features, mask, starts, ends, W1, b1r, W2, b2r)
    return out


# speedup vs baseline: 2.1114x; 1.0386x over previous
import functools

import jax
import jax.numpy as jnp
from jax.experimental import pallas as pl
from jax.experimental.pallas import tpu as pltpu


def _fused_kernel(x_ref, mask_ref, starts_ref, ends_ref, w1_ref, b1_ref,
                  w2_ref, b2_ref, out_ref):
    nb = x_ref.shape[0]
    for i in range(nb):
        x = x_ref[i] * mask_ref[i]
        h = jnp.dot(w1_ref[...], x, preferred_element_type=jnp.float32)
        h = jnp.maximum(h + b1_ref[0][:, None], 0.0)
        s = jnp.dot(w2_ref[...], h, preferred_element_type=jnp.float32)
        s = s + b2_ref[0, 0]

        t = jax.lax.broadcasted_iota(
            jnp.int32, (starts_ref.shape[-1], s.shape[-1]), 1)
        starts = starts_ref[i, 0, :][:, None]
        ends = ends_ref[i, 0, :][:, None]
        in_word = (t >= starts) & (t < ends)
        masked = jnp.where(in_word, s, -jnp.inf)
        out_ref[i, 0, :] = jnp.max(masked, axis=-1)


def kernel(features, word_bounds, word_lengths, mask, W1, b1, W2, b2):
    B, D, T = features.shape
    H = W1.shape[0]
    W = word_bounds.shape[-1]

    starts = word_bounds[:, 0, :].astype(jnp.int32).reshape(B, 1, W)
    ends = word_bounds[:, 1, :].astype(jnp.int32).reshape(B, 1, W)
    b1r = b1.reshape(1, H).astype(jnp.float32)
    b2r = b2.reshape(1, 1).astype(jnp.float32)

    NB = 2
    out = pl.pallas_call(
        _fused_kernel,
        grid=(B // NB,),
        in_specs=[
            pl.BlockSpec((NB, D, T), lambda b: (b, 0, 0)),
            pl.BlockSpec((NB, 1, T), lambda b: (b, 0, 0)),
            pl.BlockSpec((NB, 1, W), lambda b: (b, 0, 0)),
            pl.BlockSpec((NB, 1, W), lambda b: (b, 0, 0)),
            pl.BlockSpec((H, D), lambda b: (0, 0)),
            pl.BlockSpec((1, H), lambda b: (0, 0)),
            pl.BlockSpec((1, H), lambda b: (0, 0)),
            pl.BlockSpec((1, 1), lambda b: (0, 0)),
        ],
        out_specs=pl.BlockSpec((NB, 1, W), lambda b: (b, 0, 0)),
        out_shape=jax.ShapeDtypeStruct((B, 1, W), jnp.float32),
        compiler_params=pltpu.CompilerParams(
            dimension_semantics=("parallel",)),
    )(features, mask, starts, ends, W1, b1r, W2, b2r)
    return out
